# baseline (speedup 1.0000x reference)
import functools

import jax
import jax.numpy as jnp
from jax import lax
from jax.experimental import pallas as pl
from jax.experimental.pallas import tpu as pltpu
from jax.experimental.pallas import tpu_sc as plsc

NUM_TEMPS = 1000
BATCH = 16384
LANES = 16

_info = plsc.get_sparse_core_info()
_NC, _NS = 1, _info.num_subcores
_NW = _NC * _NS
_B_PER_W = BATCH // _NW
_STEPS = _B_PER_W // LANES


def _body(temps_hbm, rs_hbm, out_hbm, rs_v, idx_v, out_v, sem_rs, sem_idx,
          sem_out):
    wid = lax.axis_index("s") * _NC + lax.axis_index("c")
    base = wid * _B_PER_W

    cp_rs = pltpu.async_copy(rs_hbm, rs_v, sem_rs)
    cp_idx = pltpu.async_copy(temps_hbm.at[pl.ds(base, _B_PER_W)], idx_v,
                              sem_idx)
    cp_rs.wait()
    cp_idx.wait()

    zero = jnp.zeros((LANES,), jnp.float32)
    BLK = 8
    W = BLK * LANES
    copies = []
    for b in range(0, _STEPS, BLK):
        idxs = [idx_v[pl.ds((b + j) * LANES, LANES)] for j in range(BLK)]
        vals = [plsc.load_gather(rs_v, [idxs[j]]) for j in range(BLK)]
        for j in range(BLK):
            out_v[pl.ds((b + j) * LANES, LANES)] = jnp.maximum(vals[j], zero)
        copies.append(pltpu.async_copy(
            out_v.at[pl.ds(b * LANES, W)],
            out_hbm.at[pl.ds(base + b * LANES, W)], sem_out))
    for cp in copies:
        cp.wait()


@jax.jit
def kernel(temps, rs):
    mesh = plsc.VectorSubcoreMesh(core_axis_name="c", subcore_axis_name="s",
                                  num_cores=1)
    run = pl.kernel(
        _body,
        out_type=jax.ShapeDtypeStruct((BATCH,), jnp.float32),
        mesh=mesh,
        scratch_types=[
            pltpu.VMEM((NUM_TEMPS,), jnp.float32),
            pltpu.VMEM((_B_PER_W,), jnp.int32),
            pltpu.VMEM((_B_PER_W,), jnp.float32),
            pltpu.SemaphoreType.DMA,
            pltpu.SemaphoreType.DMA,
            pltpu.SemaphoreType.DMA,
        ],
        compiler_params=pltpu.CompilerParams(
            needs_layout_passes=False,
            skip_device_barrier=True,
            disable_bounds_checks=True,
            disable_semaphore_checks=True,
        ),
    )
    return run(temps, rs)


# speedup vs baseline: 1.0107x; 1.0107x over previous
import functools

import jax
import jax.numpy as jnp
from jax import lax
from jax.experimental import pallas as pl
from jax.experimental.pallas import tpu as pltpu
from jax.experimental.pallas import tpu_sc as plsc

NUM_TEMPS = 1000
BATCH = 16384
LANES = 16

_info = plsc.get_sparse_core_info()
_NC, _NS = 1, _info.num_subcores
_NW = _NC * _NS
_B_PER_W = BATCH // _NW
_STEPS = _B_PER_W // LANES


def _body(temps_hbm, rs_hbm, out_hbm, rs_v, idx_v, out_v, sem_rs, sem_idx,
          sem_out):
    wid = lax.axis_index("s") * _NC + lax.axis_index("c")
    base = wid * _B_PER_W

    cp_rs = pltpu.async_copy(rs_hbm, rs_v, sem_rs)
    cp_idx = pltpu.async_copy(temps_hbm.at[pl.ds(base, _B_PER_W)], idx_v,
                              sem_idx)
    cp_rs.wait()
    cp_idx.wait()

    zero = jnp.zeros((LANES,), jnp.float32)
    BLK = 8
    HALF = _B_PER_W // 2
    cp_out1 = None
    for b in range(0, _STEPS, BLK):
        idxs = [idx_v[pl.ds((b + j) * LANES, LANES)] for j in range(BLK)]
        vals = [plsc.load_gather(rs_v, [idxs[j]]) for j in range(BLK)]
        for j in range(BLK):
            out_v[pl.ds((b + j) * LANES, LANES)] = jnp.maximum(vals[j], zero)
        if (b + BLK) * LANES == HALF:
            cp_out1 = pltpu.async_copy(out_v.at[pl.ds(0, HALF)],
                                       out_hbm.at[pl.ds(base, HALF)], sem_out)

    cp_out2 = pltpu.async_copy(out_v.at[pl.ds(HALF, HALF)],
                               out_hbm.at[pl.ds(base + HALF, HALF)], sem_out)
    cp_out1.wait()
    cp_out2.wait()


@jax.jit
def kernel(temps, rs):
    mesh = plsc.VectorSubcoreMesh(core_axis_name="c", subcore_axis_name="s",
                                  num_cores=1)
    run = pl.kernel(
        _body,
        out_type=jax.ShapeDtypeStruct((BATCH,), jnp.float32),
        mesh=mesh,
        scratch_types=[
            pltpu.VMEM((NUM_TEMPS,), jnp.float32),
            pltpu.VMEM((_B_PER_W,), jnp.int32),
            pltpu.VMEM((_B_PER_W,), jnp.float32),
            pltpu.SemaphoreType.DMA,
            pltpu.SemaphoreType.DMA,
            pltpu.SemaphoreType.DMA,
        ],
        compiler_params=pltpu.CompilerParams(
            needs_layout_passes=False,
            skip_device_barrier=True,
            disable_bounds_checks=True,
            disable_semaphore_checks=True,
        ),
    )
    return run(temps, rs)
